# baseline (speedup 1.0000x reference)
import jax
import jax.numpy as jnp
from jax.experimental import pallas as pl

_HIDDEN = 4096
_E = 64
_TOPK = 8
_BT = 1024


def _router_body(w_ref, b_ref, x_ref, logits_ref, wts_ref, sel_ref, mask_ref):
    w = w_ref[...]
    x = x_ref[...]
    logits = jax.lax.dot_general(
        w, x, (((1,), (1,)), ((), ())),
        preferred_element_type=jnp.float32,
        precision=jax.lax.Precision.DEFAULT)
    logits = logits + b_ref[...]
    logits_ref[...] = logits.T

    eio = jax.lax.broadcasted_iota(jnp.int32, (_E, _BT), 0)
    work = logits
    neg_inf = jnp.float32(-jnp.inf)
    vals = []
    idxs = []
    for k in range(_TOPK):
        m = jnp.max(work, axis=0, keepdims=True)
        ismax = work == m
        idx = jnp.min(jnp.where(ismax, eio, _E), axis=0, keepdims=True)
        onehot = eio == idx
        mask_ref[:, k, :] = onehot.astype(jnp.int32)
        vals.append(m)
        idxs.append(idx)
        work = jnp.where(onehot, neg_inf, work)

    vals = jnp.concatenate(vals, axis=0)
    sel = jnp.concatenate(idxs, axis=0)
    e = jnp.exp(vals - vals[0:1])
    wts_ref[...] = (e / jnp.sum(e, axis=0, keepdims=True)).T
    sel_ref[...] = sel.astype(jnp.float32).T.astype(jnp.int32)


def _router_call(x, W, b2, interpret=False):
    T = x.shape[0]
    return pl.pallas_call(
        _router_body,
        grid=(T // _BT,),
        in_specs=[
            pl.BlockSpec((_E, _HIDDEN), lambda i: (0, 0)),
            pl.BlockSpec((_E, 1), lambda i: (0, 0)),
            pl.BlockSpec((_BT, _HIDDEN), lambda i: (i, 0)),
        ],
        out_specs=[
            pl.BlockSpec((_BT, _E), lambda i: (i, 0)),
            pl.BlockSpec((_BT, _TOPK), lambda i: (i, 0)),
            pl.BlockSpec((_BT, _TOPK), lambda i: (i, 0)),
            pl.BlockSpec((_E, _TOPK, _BT), lambda i: (0, 0, i)),
        ],
        out_shape=[
            jax.ShapeDtypeStruct((T, _E), jnp.float32),
            jax.ShapeDtypeStruct((T, _TOPK), jnp.float32),
            jax.ShapeDtypeStruct((T, _TOPK), jnp.int32),
            jax.ShapeDtypeStruct((_E, _TOPK, T), jnp.int32),
        ],
        interpret=interpret,
    )(W, b2, x)


@jax.jit
def kernel(x, W, b):
    return tuple(_router_call(x, W, b.reshape(_E, 1)))


# speedup vs baseline: 1.0446x; 1.0446x over previous
import jax
import jax.numpy as jnp
from jax.experimental import pallas as pl

_HIDDEN = 4096
_E = 64
_TOPK = 8
_BT = 1024


def _router_body(w_ref, b_ref, brow_ref, x_ref, logits_ref, wts_ref, sel_ref, mask_ref):
    w = w_ref[...]
    x = x_ref[...]
    logits = jax.lax.dot_general(
        w, x, (((1,), (1,)), ((), ())),
        preferred_element_type=jnp.float32,
        precision=jax.lax.Precision.DEFAULT)
    logits = logits + b_ref[...]
    logits_ref[...] = jax.lax.dot_general(
        x, w, (((1,), (1,)), ((), ())),
        preferred_element_type=jnp.float32,
        precision=jax.lax.Precision.DEFAULT) + brow_ref[...]

    eio = jax.lax.broadcasted_iota(jnp.int32, (_E, _BT), 0)
    work = logits
    neg_inf = jnp.float32(-jnp.inf)
    vals = []
    idxs = []
    for k in range(_TOPK):
        m = jnp.max(work, axis=0, keepdims=True)
        ismax = work == m
        idx = jnp.min(jnp.where(ismax, eio, _E), axis=0, keepdims=True)
        onehot = eio == idx
        mask_ref[:, k, :] = onehot.astype(jnp.int32)
        vals.append(m)
        idxs.append(idx)
        work = jnp.where(onehot, neg_inf, work)

    vals = jnp.concatenate(vals, axis=0)
    sel = jnp.concatenate(idxs, axis=0)
    e = jnp.exp(vals - vals[0:1])
    wts_ref[...] = e / jnp.sum(e, axis=0, keepdims=True)
    sel_ref[...] = sel


def _router_call(x, W, b2, interpret=False):
    T = x.shape[0]
    return pl.pallas_call(
        _router_body,
        grid=(T // _BT,),
        in_specs=[
            pl.BlockSpec((_E, _HIDDEN), lambda i: (0, 0)),
            pl.BlockSpec((_E, 1), lambda i: (0, 0)),
            pl.BlockSpec((1, _E), lambda i: (0, 0)),
            pl.BlockSpec((_BT, _HIDDEN), lambda i: (i, 0)),
        ],
        out_specs=[
            pl.BlockSpec((_BT, _E), lambda i: (i, 0)),
            pl.BlockSpec((_TOPK, _BT), lambda i: (0, i)),
            pl.BlockSpec((_TOPK, _BT), lambda i: (0, i)),
            pl.BlockSpec((_E, _TOPK, _BT), lambda i: (0, 0, i)),
        ],
        out_shape=[
            jax.ShapeDtypeStruct((T, _E), jnp.float32),
            jax.ShapeDtypeStruct((_TOPK, T), jnp.float32),
            jax.ShapeDtypeStruct((_TOPK, T), jnp.int32),
            jax.ShapeDtypeStruct((_E, _TOPK, T), jnp.int32),
        ],
        interpret=interpret,
    )(W, b2, b2.reshape(1, _E), x)


@jax.jit
def kernel(x, W, b):
    logits, wtsT, selT, mask = _router_call(x, W, b.reshape(_E, 1))
    return (logits, wtsT.T, selT.T, mask)


# speedup vs baseline: 1.2632x; 1.2093x over previous
import jax
import jax.numpy as jnp
from jax.experimental import pallas as pl

_HIDDEN = 4096
_E = 64
_TOPK = 8
_BT = 1024


def _router_body(w_ref, b_ref, x_ref, logits_ref, wts_ref, sel_ref, mask_ref):
    w = w_ref[...]
    x = x_ref[...]
    logits = jax.lax.dot_general(
        w, x, (((1,), (1,)), ((), ())),
        preferred_element_type=jnp.float32,
        precision=jax.lax.Precision.DEFAULT)
    logits = logits + b_ref[...]
    logits_ref[...] = logits

    eio = jax.lax.broadcasted_iota(jnp.int32, (_E, _BT), 0)
    work = logits
    neg_inf = jnp.float32(-jnp.inf)
    vals = []
    idxs = []
    for k in range(_TOPK):
        m = jnp.max(work, axis=0, keepdims=True)
        ismax = work == m
        idx = jnp.min(jnp.where(ismax, eio, _E), axis=0, keepdims=True)
        onehot = eio == idx
        mask_ref[:, k, :] = onehot.astype(jnp.int32)
        vals.append(m)
        idxs.append(idx)
        work = jnp.where(onehot, neg_inf, work)

    vals = jnp.concatenate(vals, axis=0)
    sel = jnp.concatenate(idxs, axis=0)
    e = jnp.exp(vals - vals[0:1])
    wts_ref[...] = e / jnp.sum(e, axis=0, keepdims=True)
    sel_ref[...] = sel


def _router_call(x, W, b2, interpret=False):
    T = x.shape[0]
    return pl.pallas_call(
        _router_body,
        grid=(T // _BT,),
        in_specs=[
            pl.BlockSpec((_E, _HIDDEN), lambda i: (0, 0)),
            pl.BlockSpec((_E, 1), lambda i: (0, 0)),
            pl.BlockSpec((_BT, _HIDDEN), lambda i: (i, 0)),
        ],
        out_specs=[
            pl.BlockSpec((_E, _BT), lambda i: (0, i)),
            pl.BlockSpec((_TOPK, _BT), lambda i: (0, i)),
            pl.BlockSpec((_TOPK, _BT), lambda i: (0, i)),
            pl.BlockSpec((_E, _TOPK, _BT), lambda i: (0, 0, i)),
        ],
        out_shape=[
            jax.ShapeDtypeStruct((_E, T), jnp.float32),
            jax.ShapeDtypeStruct((_TOPK, T), jnp.float32),
            jax.ShapeDtypeStruct((_TOPK, T), jnp.int32),
            jax.ShapeDtypeStruct((_E, _TOPK, T), jnp.int32),
        ],
        interpret=interpret,
    )(W, b2, x)


@jax.jit
def kernel(x, W, b):
    logitsT, wtsT, selT, mask = _router_call(x, W, b.reshape(_E, 1))
    return (logitsT.T, wtsT.T, selT.T, mask)
